# baseline (speedup 1.0000x reference)
import jax
import jax.numpy as jnp
from jax import lax
from jax.experimental import pallas as pl
from jax.experimental.pallas import tpu as pltpu
from jax.experimental.pallas import tpu_sc as plsc

ROWS, COLS = 4096, 200
EMBED = 64
VOCAB = 1000000
NC, NS = 2, 16
NW = NC * NS
IBLK = ROWS // NW
PITCH = 136

VBLK = 256
SPITCH = 264
FULL = (VOCAB // 128) * 128
LEFT = (VOCAB // 128) % ((VBLK // 128) * NW)

_params = pltpu.CompilerParams(use_tc_tiling_on_sc=True,
                               needs_layout_passes=False)


def _repack_body(tabt_hbm, tail_hbm, pairs_hbm, src_v, dst_v, tail_v,
                 isem0, isem1, osem0, osem1):
    w = lax.axis_index("s") * NC + lax.axis_index("c")
    isems = (isem0, isem1)
    osems = (osem0, osem1)
    lanes = lax.iota(jnp.int32, 16)
    evecs = [lanes + 16 * t for t in range(4)]
    nround = (VOCAB // 128 - LEFT) // ((VBLK // 128) * NW)

    def start_in(r, b):
        v0 = pl.multiple_of((r * NW + w) * VBLK, VBLK)
        pltpu.async_copy(tabt_hbm.at[:, pl.ds(v0, VBLK)],
                         src_v.at[b, :, pl.ds(0, VBLK)], isems[b])

    def transpose_panel(b, width):
        @plsc.parallel_loop(0, width // 2, unroll=4)
        def _(p):
            cols = [jnp.broadcast_to(2 * p + q, (16,)).astype(jnp.int32)
                    for q in range(2)]
            for cg in range(8):
                v = plsc.load_gather(src_v.at[b], [evecs[cg % 4], cols[cg // 4]])
                dst_v[b, p, pl.ds(16 * cg, 16)] = v

    start_in(0, 0)

    def step(i, carry):
        for b in range(2):
            r = i * 2 + b

            @pl.when(r < nround - 1)
            def _():
                start_in(r + 1, 1 - b)

            pltpu.make_async_copy(tabt_hbm.at[:, pl.ds(0, VBLK)],
                                  src_v.at[b, :, pl.ds(0, VBLK)],
                                  isems[b]).wait()

            @pl.when(r >= 2)
            def _():
                pltpu.make_async_copy(dst_v.at[b], pairs_hbm.at[pl.ds(0, VBLK // 2)],
                                      osems[b]).wait()

            transpose_panel(b, VBLK)
            p0 = pl.multiple_of((r * NW + w) * (VBLK // 2), VBLK // 2)
            pltpu.async_copy(dst_v.at[b], pairs_hbm.at[pl.ds(p0, VBLK // 2)],
                             osems[b])
        return carry

    lax.fori_loop(0, nround // 2, step, 0, unroll=False)
    for b in range(2):
        pltpu.make_async_copy(dst_v.at[b], pairs_hbm.at[pl.ds(0, VBLK // 2)],
                              osems[b]).wait()

    @pl.when(w < LEFT)
    def _():
        v0 = pl.multiple_of((VOCAB // 128 - LEFT + w) * 128, 128)
        pltpu.sync_copy(tabt_hbm.at[:, pl.ds(v0, 128)],
                        src_v.at[0, :, pl.ds(0, 128)])
        transpose_panel(0, 128)
        pltpu.sync_copy(dst_v.at[0, pl.ds(0, 64)],
                        pairs_hbm.at[pl.ds(pl.multiple_of(v0 // 2, 64), 64)])

    @pl.when(w == NW - 1)
    def _():
        pltpu.sync_copy(tail_hbm, tail_v)

        @plsc.parallel_loop(0, 32, unroll=4)
        def _(p):
            cols = [jnp.broadcast_to(2 * p + q, (16,)).astype(jnp.int32)
                    for q in range(2)]
            for cg in range(8):
                v = plsc.load_gather(tail_v, [evecs[cg % 4], cols[cg // 4]])
                dst_v[0, p, pl.ds(16 * cg, 16)] = v

        pltpu.sync_copy(dst_v.at[0, pl.ds(0, 32)],
                        pairs_hbm.at[pl.ds(FULL // 2, 32)])


@jax.jit
def _repack(tabt, tail_t):
    mesh = plsc.VectorSubcoreMesh(core_axis_name="c", subcore_axis_name="s")
    fn = pl.kernel(
        _repack_body,
        out_type=jax.ShapeDtypeStruct((VOCAB // 2, 128), jnp.float32),
        mesh=mesh,
        scratch_types=[
            pltpu.VMEM((2, EMBED, SPITCH), jnp.float32),
            pltpu.VMEM((2, VBLK // 2, 128), jnp.float32),
            pltpu.VMEM((EMBED, 64), jnp.float32),
            pltpu.SemaphoreType.DMA,
            pltpu.SemaphoreType.DMA,
            pltpu.SemaphoreType.DMA,
            pltpu.SemaphoreType.DMA,
        ],
        compiler_params=_params,
    )
    return fn(tabt, tail_t)


def _gather_body(idx_hbm, tab_hbm, out_hbm, idx_v, pidx_v, rows_v, blk_v,
                 gsem0, gsem1, osem0, osem1):
    w = lax.axis_index("s") * NC + lax.axis_index("c")
    gsems = (gsem0, gsem1)
    osems = (osem0, osem1)
    lanes = lax.iota(jnp.int32, 16)
    evecs = [lanes + 16 * t for t in range(EMBED // 16)]

    pltpu.sync_copy(idx_hbm.at[:, pl.ds(pl.multiple_of(w * IBLK, IBLK), IBLK)],
                    idx_v)

    def prep(j, b):
        for g in range(IBLK // 16):
            t = idx_v[j, pl.ds(16 * g, 16)]
            pidx_v[b, pl.ds(16 * g, 16)] = lax.shift_right_logical(t, 1)

    def start_gather(b):
        pltpu.async_copy(tab_hbm.at[pidx_v.at[b]], rows_v.at[b], gsems[b])

    prep(0, 0)
    start_gather(0)

    def step(i, carry):
        for b in range(2):
            j = i * 2 + b

            @pl.when(j < COLS - 1)
            def _():
                prep(j + 1, 1 - b)
                start_gather(1 - b)

            pltpu.make_async_copy(tab_hbm.at[pl.ds(0, IBLK)],
                                  rows_v.at[b], gsems[b]).wait()

            @pl.when(j >= 2)
            def _():
                pltpu.make_async_copy(blk_v.at[b, :, :, pl.ds(0, 128)],
                                      out_hbm.at[0, :, 0, :, :],
                                      osems[b]).wait()

            @plsc.parallel_loop(0, IBLK // 16, unroll=2)
            def _(g):
                qvec = (idx_v[j, pl.ds(g * 16, 16)] & 1) * EMBED
                for m in range(16):
                    q = qvec[m]
                    k = g * 16 + m
                    kv = jnp.broadcast_to(k, (16,)).astype(jnp.int32)
                    for t in range(EMBED // 16):
                        v = rows_v[b, k, pl.ds(q + 16 * t, 16)]
                        plsc.store_scatter(
                            blk_v.at[b],
                            [lax.shift_right_logical(evecs[t], 3),
                             evecs[t] & 7, kv], v)

            pltpu.async_copy(blk_v.at[b, :, :, pl.ds(0, 128)],
                             out_hbm.at[j, :, w, :, :], osems[b])
        return carry

    lax.fori_loop(0, COLS // 2, step, 0, unroll=False)

    for b in range(2):
        pltpu.make_async_copy(blk_v.at[b, :, :, pl.ds(0, 128)],
                              out_hbm.at[0, :, 0, :, :], osems[b]).wait()


@jax.jit
def _embed(idx_t, tab_pairs):
    mesh = plsc.VectorSubcoreMesh(core_axis_name="c", subcore_axis_name="s")
    fn = pl.kernel(
        _gather_body,
        out_type=jax.ShapeDtypeStruct((COLS, 8, NW, 8, 128), jnp.float32),
        mesh=mesh,
        scratch_types=[
            pltpu.VMEM((COLS, IBLK), jnp.int32),
            pltpu.VMEM((2, IBLK), jnp.int32),
            pltpu.VMEM((2, IBLK, 128), jnp.float32),
            pltpu.VMEM((2, 8, 8, PITCH), jnp.float32),
            pltpu.SemaphoreType.DMA,
            pltpu.SemaphoreType.DMA,
            pltpu.SemaphoreType.DMA,
            pltpu.SemaphoreType.DMA,
        ],
        compiler_params=_params,
    )
    return fn(idx_t, tab_pairs)


def kernel(token_idxs, table):
    idx_t = token_idxs.T
    tabt = table.T
    tail_t = table[FULL:].T
    pairs = _repack(tabt, tail_t)
    out5 = _embed(idx_t, pairs)
    return out5.transpose(2, 4, 0, 1, 3).reshape(ROWS, COLS, EMBED)


# speedup vs baseline: 1.2947x; 1.2947x over previous
import jax
import jax.numpy as jnp
from jax import lax
from jax.experimental import pallas as pl
from jax.experimental.pallas import tpu as pltpu
from jax.experimental.pallas import tpu_sc as plsc

ROWS, COLS = 4096, 200
EMBED = 64
B = ROWS * COLS
NC, NS = 2, 16
NW = NC * NS
B_PER_W = B // NW
CHUNK = 800
NCHUNK = B_PER_W // CHUNK


def _gather_body(idx_hbm, table_hbm, out_hbm, idx_v, rows_v, gsem0, gsem1):
    wid = lax.axis_index("s") * NC + lax.axis_index("c")
    base = wid * B_PER_W
    gsems = (gsem0, gsem1)

    pltpu.sync_copy(idx_hbm.at[pl.ds(pl.multiple_of(base, B_PER_W), B_PER_W)],
                    idx_v)

    def start_gather(g, b):
        off = pl.multiple_of(g * CHUNK, CHUNK)
        pltpu.async_copy(table_hbm.at[idx_v.at[pl.ds(off, CHUNK)]],
                         rows_v.at[b], gsems[b])

    start_gather(0, 0)

    def step(i, carry):
        for b in range(2):
            g = i * 2 + b
            pltpu.make_async_copy(table_hbm.at[pl.ds(0, CHUNK)],
                                  rows_v.at[b], gsems[b]).wait()

            @pl.when(g < NCHUNK - 1)
            def _():
                start_gather(g + 1, 1 - b)

            off = pl.multiple_of(base + g * CHUNK, CHUNK)
            pltpu.sync_copy(rows_v.at[b], out_hbm.at[pl.ds(off, CHUNK)])
        return carry

    lax.fori_loop(0, NCHUNK // 2, step, 0, unroll=False)


@jax.jit
def _embed(idx_flat, table):
    mesh = plsc.VectorSubcoreMesh(core_axis_name="c", subcore_axis_name="s")
    fn = pl.kernel(
        _gather_body,
        out_type=jax.ShapeDtypeStruct((B, EMBED), jnp.float32),
        mesh=mesh,
        scratch_types=[
            pltpu.VMEM((B_PER_W,), jnp.int32),
            pltpu.VMEM((2, CHUNK, EMBED), jnp.float32),
            pltpu.SemaphoreType.DMA,
            pltpu.SemaphoreType.DMA,
        ],
        compiler_params=pltpu.CompilerParams(use_tc_tiling_on_sc=False),
    )
    return fn(idx_flat, table)


def kernel(token_idxs, table):
    idx_flat = token_idxs.reshape(B).astype(jnp.int32)
    out = _embed(idx_flat, table)
    return out.reshape(ROWS, COLS, EMBED)


# speedup vs baseline: 1.7206x; 1.3290x over previous
import jax
import jax.numpy as jnp
from jax import lax
from jax.experimental import pallas as pl
from jax.experimental.pallas import tpu as pltpu
from jax.experimental.pallas import tpu_sc as plsc

ROWS, COLS = 4096, 200
EMBED = 64
B = ROWS * COLS
NC, NS = 2, 16
NW = NC * NS
B_PER_W = B // NW
CHUNK = 800
NCHUNK = B_PER_W // CHUNK


def _gather_body(idx_hbm, table_hbm, out_hbm, idx_v, rows_v, gsem0, gsem1):
    wid = lax.axis_index("s") * NC + lax.axis_index("c")
    base = wid * B_PER_W
    gsems = (gsem0, gsem1)

    pltpu.sync_copy(idx_hbm.at[pl.ds(pl.multiple_of(base, B_PER_W), B_PER_W)],
                    idx_v)

    def start_gather(g, b):
        off = pl.multiple_of(g * CHUNK, CHUNK)
        pltpu.async_copy(table_hbm.at[idx_v.at[pl.ds(off, CHUNK)]],
                         rows_v.at[b], gsems[b])

    start_gather(0, 0)

    def step(i, carry):
        for b in range(2):
            g = i * 2 + b
            pltpu.make_async_copy(table_hbm.at[pl.ds(0, CHUNK)],
                                  rows_v.at[b], gsems[b]).wait()

            @pl.when(g < NCHUNK - 1)
            def _():
                start_gather(g + 1, 1 - b)

            off = pl.multiple_of(base + g * CHUNK, CHUNK)
            pltpu.sync_copy(rows_v.at[b],
                            out_hbm.at[pl.ds(off, CHUNK), pl.ds(0, EMBED)])
        return carry

    lax.fori_loop(0, NCHUNK // 2, step, 0, unroll=False)


@jax.jit
def _embed(idx_flat, table):
    mesh = plsc.VectorSubcoreMesh(core_axis_name="c", subcore_axis_name="s")
    fn = pl.kernel(
        _gather_body,
        out_type=jax.ShapeDtypeStruct((B, 128), jnp.float32),
        mesh=mesh,
        scratch_types=[
            pltpu.VMEM((B_PER_W,), jnp.int32),
            pltpu.VMEM((2, CHUNK, EMBED), jnp.float32),
            pltpu.SemaphoreType.DMA,
            pltpu.SemaphoreType.DMA,
        ],
        compiler_params=pltpu.CompilerParams(use_tc_tiling_on_sc=False),
    )
    return fn(idx_flat, table)


def kernel(token_idxs, table):
    idx_flat = token_idxs.reshape(B).astype(jnp.int32)
    out = _embed(idx_flat, table)
    return out.reshape(ROWS, COLS, 128)[:, :, :EMBED]
